# Initial kernel scaffold; baseline (speedup 1.0000x reference)
import jax
import jax.numpy as jnp
from jax.experimental import pallas as pl


def kernel(x, edge_index, W_gat, att_src, att_dst, bias_gat, conv_w, conv_b, lin_w, lin_b):
    raise NotImplementedError("write your pallas kernel here")


# speedup vs baseline: 4.5819x; 4.5819x over previous
import functools

import jax
import jax.numpy as jnp
from jax import lax
from jax.experimental import pallas as pl
from jax.experimental.pallas import tpu as pltpu, tpu_sc as plsc


def _k1_body(x_ref, w_ref, atts_ref, attd_ref, e8_ref, h2_ref, asrc_ref, adst_ref, hs_ref):
    h = jnp.dot(x_ref[...], w_ref[...], preferred_element_type=jnp.float32, precision=jax.lax.Precision.HIGHEST)
    e8 = e8_ref[...]
    asrc_ref[...] = jnp.dot(h * atts_ref[...], e8, preferred_element_type=jnp.float32, precision=jax.lax.Precision.HIGHEST)
    adst_ref[...] = jnp.dot(h * attd_ref[...], e8, preferred_element_type=jnp.float32, precision=jax.lax.Precision.HIGHEST)
    hs_ref[...] = jnp.dot(h, e8, preferred_element_type=jnp.float32, precision=jax.lax.Precision.HIGHEST)
    h2_ref[...] = h


def _k5_body(fuse_ref, x_ref, cb_ref, w_ref, b_ref, out_ref):
    hf = jnp.maximum(fuse_ref[...] + cb_ref[...] + x_ref[...], 0.0)
    out_ref[...] = jnp.dot(hf, w_ref[...], preferred_element_type=jnp.float32, precision=jax.lax.Precision.HIGHEST) + b_ref[...]


def _k2_body(E_PAD, E_ALL, srci, dsti, asrc, adst, hst, zn,
             ex_out, hss_out, denp_out,
             sidx, didx, gsa, gda, ghs, exb, den):
    cid = lax.axis_index("c")
    sid = lax.axis_index("s")
    wid = cid * 16 + sid
    q = E_PAD // 32
    base = wid * q
    it = lax.iota(jnp.int32, 16)
    row0 = it // 8
    col = it % 8

    @pl.when(sid == 0)
    def _():
        pltpu.sync_copy(zn, den)

    for r in range(16):
        exb[r, :] = jnp.zeros((16,), jnp.float32)
    plsc.subcore_barrier()

    @pl.loop(0, q // 16)
    def _(j):
        off = base + j * 16
        pltpu.sync_copy(srci.at[pl.ds(off, 16)], sidx)
        pltpu.sync_copy(dsti.at[pl.ds(off, 16)], didx)
        pltpu.sync_copy(asrc.at[sidx], gsa)
        pltpu.sync_copy(adst.at[didx], gda)
        pltpu.sync_copy(hst.at[sidx], ghs)
        for p in range(8):
            rowp = row0 + 2 * p
            va = plsc.load_gather(gsa, [rowp, col])
            vd = plsc.load_gather(gda, [rowp, col])
            al = va + vd
            al = jnp.maximum(al, 0.2 * al)
            ex = jnp.exp(al)
            glob = off + 2 * p + row0
            ex = jnp.where(glob < E_ALL, ex, 0.0)
            plsc.store_scatter(exb, [rowp, col], ex)
        pltpu.sync_copy(exb, ex_out.at[pl.ds(off, 16)])
        pltpu.sync_copy(ghs, hss_out.at[pl.ds(off, 16)])
        pltpu.sync_copy(exb, den.at[didx], add=True)

    plsc.subcore_barrier()

    @pl.when(sid == 0)
    def _():
        pltpu.sync_copy(den, denp_out.at[cid])


def _k3_body(E_PAD, ex_in, hss_in, dsti, den0, den1,
             coef_out, ms_out,
             didx, exb, hsb, g0, g1, cfb, msb):
    cid = lax.axis_index("c")
    sid = lax.axis_index("s")
    wid = cid * 16 + sid
    q = E_PAD // 32
    base = wid * q
    it = lax.iota(jnp.int32, 16)
    row0 = it // 8
    col = it % 8

    acc_fin = pl.loop(0, q // 16, init_carry=jnp.zeros((16,), jnp.float32))(
        lambda j, acc: _k3_chunk(base + j * 16, acc, ex_in, hss_in, dsti, den0, den1,
                                 coef_out, didx, exb, hsb, g0, g1, cfb, row0, col))
    msb[...] = acc_fin
    pltpu.sync_copy(msb, ms_out.at[wid])


def _k3_chunk(off, acc, ex_in, hss_in, dsti, den0, den1, coef_out,
              didx, exb, hsb, g0, g1, cfb, row0, col):
    pltpu.sync_copy(dsti.at[pl.ds(off, 16)], didx)
    pltpu.sync_copy(ex_in.at[pl.ds(off, 16)], exb)
    pltpu.sync_copy(hss_in.at[pl.ds(off, 16)], hsb)
    pltpu.sync_copy(den0.at[didx], g0)
    pltpu.sync_copy(den1.at[didx], g1)
    for p in range(8):
        rowp = row0 + 2 * p
        ve = plsc.load_gather(exb, [rowp, col])
        vh = plsc.load_gather(hsb, [rowp, col])
        v0 = plsc.load_gather(g0, [rowp, col])
        v1 = plsc.load_gather(g1, [rowp, col])
        cf = ve / (v0 + v1 + 1e-16)
        acc = acc + cf * vh
        plsc.store_scatter(cfb, [rowp, col], cf)
    pltpu.sync_copy(cfb, coef_out.at[pl.ds(off, 16)])
    return acc


def _k4_body(E_PAD, N, B, coef, gsrc32, dsti, h3f, st, fuse_out,
             sidxv, didxv, hb, cfb, stb, facc, sem):
    cid = lax.axis_index("c")
    sid = lax.axis_index("s")
    wid = cid * 16 + sid
    it = lax.iota(jnp.int32, 16)
    row0 = it // 8
    col8 = it % 8
    z16 = jnp.zeros((16,), jnp.float32)

    pltpu.sync_copy(st, stb)

    @pl.loop(0, N * 8 // 16)
    def _(i):
        facc[pl.ds(i * 16, 16)] = z16

    sv = [stb[h, :] for h in range(8)]

    @pl.loop(0, E_PAD // B)
    def _(j):
        off = j * B
        pltpu.sync_copy(gsrc32.at[wid].at[pl.ds(off, B)], sidxv)
        pltpu.sync_copy(dsti.at[pl.ds(off, B)], didxv)
        cp = pltpu.async_copy(h3f.at[sidxv], hb, sem)
        pltpu.sync_copy(coef.at[pl.ds(off, B)], cfb)
        cp.wait()

        @pl.loop(0, B // 16)
        def _(g):
            for kk in range(8):
                rp = g * 16 + 2 * kk + row0
                acc = z16
                for h in range(8):
                    uv = plsc.load_gather(cfb, [rp, jnp.full((16,), h, jnp.int32)]) * sv[h]
                    hv = plsc.load_gather(hb, [rp, col8 + 8 * h])
                    acc = acc + uv * hv
                dsel = plsc.load_gather(didxv, [rp])
                plsc.addupdate_scatter(facc, [dsel * 8 + col8], acc)

    pltpu.sync_copy(facc, fuse_out.at[wid])


def kernel(x, edge_index, W_gat, att_src, att_dst, bias_gat, conv_w, conv_b, lin_w, lin_b):
    N, IN_C = x.shape
    E = edge_index.shape[1]
    HEADS = att_src.shape[1]
    C = att_src.shape[2]
    HC = HEADS * C
    CH = C // 2
    HALF = HEADS * CH
    E_ALL = E + N
    E_PAD = ((E_ALL + 511) // 512) * 512
    OUT = lin_w.shape[0]

    f32 = jnp.float32
    loop = jnp.arange(N, dtype=edge_index.dtype)
    pad = jnp.zeros((E_PAD - E_ALL,), dtype=edge_index.dtype)
    src_p = jnp.concatenate([edge_index[0], loop, pad])
    dst_p = jnp.concatenate([edge_index[1], loop, pad])

    att_s = att_src.reshape(1, HC)
    att_d = att_dst.reshape(1, HC)
    e8 = jnp.repeat(jnp.eye(HEADS, dtype=f32), C, axis=0)

    BLK = 400
    h2, asrc8, adst8, hs8 = pl.pallas_call(
        _k1_body,
        grid=(N // BLK,),
        in_specs=[
            pl.BlockSpec((BLK, IN_C), lambda i: (i, 0)),
            pl.BlockSpec((IN_C, HC), lambda i: (0, 0)),
            pl.BlockSpec((1, HC), lambda i: (0, 0)),
            pl.BlockSpec((1, HC), lambda i: (0, 0)),
            pl.BlockSpec((HC, HEADS), lambda i: (0, 0)),
        ],
        out_specs=[
            pl.BlockSpec((BLK, HC), lambda i: (i, 0)),
            pl.BlockSpec((BLK, HEADS), lambda i: (i, 0)),
            pl.BlockSpec((BLK, HEADS), lambda i: (i, 0)),
            pl.BlockSpec((BLK, HEADS), lambda i: (i, 0)),
        ],
        out_shape=[
            jax.ShapeDtypeStruct((N, HC), f32),
            jax.ShapeDtypeStruct((N, HEADS), f32),
            jax.ShapeDtypeStruct((N, HEADS), f32),
            jax.ShapeDtypeStruct((N, HEADS), f32),
        ],
    )(x, W_gat, att_s, att_d, e8)

    zpad = jnp.zeros((N, 16 - HEADS), f32)
    asrc_t = jnp.concatenate([asrc8, zpad], axis=1)
    adst_t = jnp.concatenate([adst8, zpad], axis=1)
    hs_t = jnp.concatenate([hs8, zpad], axis=1)
    zn = jnp.zeros((N, 16), f32)

    mesh = plsc.VectorSubcoreMesh(core_axis_name="c", subcore_axis_name="s")

    ex_a, hss_a, denp = pl.kernel(
        functools.partial(_k2_body, E_PAD, E_ALL),
        out_type=[
            jax.ShapeDtypeStruct((E_PAD, 16), f32),
            jax.ShapeDtypeStruct((E_PAD, 16), f32),
            jax.ShapeDtypeStruct((2, N, 16), f32),
        ],
        mesh=mesh,
        compiler_params=pltpu.CompilerParams(use_tc_tiling_on_sc=False, needs_layout_passes=False),
        scratch_types=[
            pltpu.VMEM((16,), jnp.int32),
            pltpu.VMEM((16,), jnp.int32),
            pltpu.VMEM((16, 16), f32),
            pltpu.VMEM((16, 16), f32),
            pltpu.VMEM((16, 16), f32),
            pltpu.VMEM((16, 16), f32),
            pltpu.VMEM_SHARED((N, 16), f32),
        ],
    )(src_p, dst_p, asrc_t, adst_t, hs_t, zn)

    coef_a, ms = pl.kernel(
        functools.partial(_k3_body, E_PAD),
        out_type=[
            jax.ShapeDtypeStruct((E_PAD, 16), f32),
            jax.ShapeDtypeStruct((32, 16), f32),
        ],
        mesh=mesh,
        compiler_params=pltpu.CompilerParams(use_tc_tiling_on_sc=False, needs_layout_passes=False),
        scratch_types=[
            pltpu.VMEM((16,), jnp.int32),
            pltpu.VMEM((16, 16), f32),
            pltpu.VMEM((16, 16), f32),
            pltpu.VMEM((16, 16), f32),
            pltpu.VMEM((16, 16), f32),
            pltpu.VMEM((16, 16), f32),
            pltpu.VMEM((16,), f32),
        ],
    )(ex_a, hss_a, dst_p, denp[0], denp[1])

    ms2 = ms.sum(0)
    msum = ms2[:HEADS] + ms2[HEADS:2 * HEADS]
    bias2 = bias_gat.reshape(HEADS, C)
    mean = msum / (N * C) + bias2.mean(axis=1)
    t = jnp.maximum(conv_w[0, 0, 0, 0] * mean + conv_b[0], 0.0)
    s = jax.nn.softmax(t)
    st = jnp.tile(s[:, None], (1, 16)).astype(f32)
    cb = (s[:, None] * bias2).sum(0)

    h3f = h2.reshape(N, HEADS, 32, 8).transpose(2, 0, 1, 3).reshape(32 * N, 64)
    woff = (jnp.arange(32, dtype=jnp.int32) * N)[:, None]
    gsrc32 = woff + src_p[None, :]
    B4 = 256

    fuse2 = pl.kernel(
        functools.partial(_k4_body, E_PAD, N, B4),
        out_type=jax.ShapeDtypeStruct((32, N * 8), f32),
        mesh=mesh,
        compiler_params=pltpu.CompilerParams(use_tc_tiling_on_sc=False, needs_layout_passes=False),
        scratch_types=[
            pltpu.VMEM((B4,), jnp.int32),
            pltpu.VMEM((B4,), jnp.int32),
            pltpu.VMEM((B4, 64), f32),
            pltpu.VMEM((B4, 16), f32),
            pltpu.VMEM((8, 16), f32),
            pltpu.VMEM((N * 8,), f32),
            pltpu.SemaphoreType.DMA,
        ],
    )(coef_a, gsrc32, dst_p, h3f, st)

    fuse = fuse2.reshape(32, N, 8).transpose(1, 0, 2).reshape(N, C)

    out = pl.pallas_call(
        _k5_body,
        grid=(N // BLK,),
        in_specs=[
            pl.BlockSpec((BLK, C), lambda i: (i, 0)),
            pl.BlockSpec((BLK, IN_C), lambda i: (i, 0)),
            pl.BlockSpec((1, C), lambda i: (0, 0)),
            pl.BlockSpec((IN_C, OUT), lambda i: (0, 0)),
            pl.BlockSpec((1, OUT), lambda i: (0, 0)),
        ],
        out_specs=pl.BlockSpec((BLK, OUT), lambda i: (i, 0)),
        out_shape=jax.ShapeDtypeStruct((N, OUT), f32),
    )(fuse, x, cb.reshape(1, C), lin_w.T, lin_b.reshape(1, OUT))

    return out[None]
